# baseline (speedup 1.0000x reference)
import functools

import numpy as np

import jax
import jax.numpy as jnp
from jax import lax
from jax.experimental import pallas as pl
from jax.experimental.pallas import tpu as pltpu
from jax.experimental.pallas import tpu_sc as plsc

N = 10000
E = 320000
D = 128
NC = 2
NS = 16
NW = NC * NS
CHUNK = 100
G = 10
NG = 10
CH = NG * G
EPT = CH * CHUNK
assert EPT * NW == E
AR = 10240
RPT = AR // NS
TAIL = N - (NS - 1) * RPT

@functools.cache
def _sc_aggregate_fn():
    mesh = plsc.VectorSubcoreMesh(core_axis_name="c", subcore_axis_name="s")

    @functools.partial(
        pl.kernel,
        mesh=mesh,
        out_type=(jax.ShapeDtypeStruct((AR, D), jnp.float32),
                  jax.ShapeDtypeStruct((AR, D), jnp.float32)),
        scratch_types=[
            pltpu.VMEM((2, G, CHUNK), jnp.int32),
            pltpu.VMEM((2, G, CHUNK), jnp.int32),
            pltpu.VMEM((CHUNK, D), jnp.float32),
            pltpu.VMEM((CHUNK, D), jnp.float32),
            pltpu.VMEM_SHARED((AR, D), jnp.float32),
            pltpu.SemaphoreType.DMA,
            pltpu.SemaphoreType.DMA,
            pltpu.SemaphoreType.DMA,
            pltpu.SemaphoreType.DMA,
        ],
    )
    def _sc_aggregate(table_hbm, ei_hbm, zeros_hbm, out0_hbm, out1_hbm,
                      src_idx, dst_idx, rows0, rows1, acc,
                      gsem0, gsem1, isem_s, isem_d):
        c = lax.axis_index("c")
        s = lax.axis_index("s")
        wid = c * NS + s
        base = s * RPT

        @pl.when(jnp.logical_and(c == 0, s < NS - 1))
        def _():
            pltpu.sync_copy(table_hbm.at[pl.ds(base, RPT)],
                            acc.at[pl.ds(base, RPT)])

        @pl.when(jnp.logical_and(c == 0, s == NS - 1))
        def _():
            pltpu.sync_copy(table_hbm.at[pl.ds(base, TAIL)],
                            acc.at[pl.ds(base, TAIL)])
            pltpu.sync_copy(zeros_hbm.at[pl.ds(0, RPT - TAIL)],
                            acc.at[pl.ds(base + TAIL, RPT - TAIL)])

        @pl.when(c == 1)
        def _():
            pltpu.sync_copy(zeros_hbm, acc.at[pl.ds(base, RPT)])

        pltpu.sync_copy(ei_hbm.at[0, wid, 0], src_idx.at[0])
        pltpu.sync_copy(ei_hbm.at[1, wid, 0], dst_idx.at[0])
        plsc.subcore_barrier()

        pltpu.async_copy(table_hbm.at[src_idx.at[0, 0]], rows0, gsem0)
        pltpu.async_copy(table_hbm.at[src_idx.at[0, 1]], rows1, gsem1)

        def outer(g, carry):
            pg = g % 2
            png = (g + 1) % 2

            @pl.when(g + 1 < NG)
            def _():
                pltpu.async_copy(ei_hbm.at[0, wid, g + 1], src_idx.at[png],
                                 isem_s)
                pltpu.async_copy(ei_hbm.at[1, wid, g + 1], dst_idx.at[png],
                                 isem_d)

            for k in range(G):
                rows, gsem = (rows0, gsem0) if k % 2 == 0 else (rows1, gsem1)
                pltpu.make_async_copy(
                    table_hbm.at[src_idx.at[pg, k]], rows, gsem).wait()
                pltpu.sync_copy(rows, acc.at[dst_idx.at[pg, k]], add=True)
                if k < G - 2:
                    pltpu.async_copy(
                        table_hbm.at[src_idx.at[pg, k + 2]], rows, gsem)
                else:
                    @pl.when(g + 1 < NG)
                    def _(k=k, rows=rows, gsem=gsem, png=png):
                        if k == G - 2:
                            pltpu.make_async_copy(
                                ei_hbm.at[0, wid, g + 1], src_idx.at[png],
                                isem_s).wait()
                            pltpu.make_async_copy(
                                ei_hbm.at[1, wid, g + 1], dst_idx.at[png],
                                isem_d).wait()
                        pltpu.async_copy(
                            table_hbm.at[src_idx.at[png, k - (G - 2)]],
                            rows, gsem)

            return carry

        lax.fori_loop(0, NG, outer, 0)
        plsc.subcore_barrier()

        @pl.when(c == 0)
        def _():
            pltpu.sync_copy(acc.at[pl.ds(base, RPT)],
                            out0_hbm.at[pl.ds(base, RPT)])

        @pl.when(c == 1)
        def _():
            pltpu.sync_copy(acc.at[pl.ds(base, RPT)],
                            out1_hbm.at[pl.ds(base, RPT)])

    return _sc_aggregate


def _tc_mlp_bn(p0_ref, p1_ref, wa_ref, ba_ref, wb_ref, bb_ref, g_ref, be_ref,
               out_ref):
    h = p0_ref[...][:N] + p1_ref[...][:N]
    h = jnp.maximum(jnp.dot(h, wa_ref[...],
                            preferred_element_type=jnp.float32) + ba_ref[...], 0.0)
    h = jnp.dot(h, wb_ref[...], preferred_element_type=jnp.float32) + bb_ref[...]
    h = jnp.maximum(h, 0.0)
    mean = jnp.mean(h, axis=0, keepdims=True)
    ctr = h - mean
    var = jnp.mean(ctr * ctr, axis=0, keepdims=True)
    out_ref[...] = ctr * lax.rsqrt(var + 1e-5) * g_ref[...] + be_ref[...]


_tc_call = pl.pallas_call(
    _tc_mlp_bn,
    out_shape=jax.ShapeDtypeStruct((N, D), jnp.float32),
)


def _tc_mlp_bn_narrow(p0_ref, p1_ref, wa_ref, ba_ref, wb_ref, bb_ref, g_ref,
                      be_ref, out_ref):
    h = p0_ref[...][:N] + p1_ref[...][:N]
    h = jnp.maximum(jnp.dot(h, wa_ref[...],
                            preferred_element_type=jnp.float32) + ba_ref[...], 0.0)
    h = jnp.dot(h, wb_ref[...], preferred_element_type=jnp.float32) + bb_ref[...]
    h = jnp.maximum(h, 0.0)
    mean = jnp.mean(h, axis=0, keepdims=True)
    ctr = h - mean
    var = jnp.mean(ctr * ctr, axis=0, keepdims=True)
    out = ctr * lax.rsqrt(var + 1e-5) * g_ref[...] + be_ref[...]
    out_ref[...] = out[:, : out_ref.shape[1]]


_tc_call2 = pl.pallas_call(
    _tc_mlp_bn_narrow,
    out_shape=jax.ShapeDtypeStruct((N, 2), jnp.float32),
)


_ZEROS = np.zeros((RPT, D), np.float32)


def kernel(x_in, edge_index_in, W1, b1, W2, b2, W3, b3, W4, b4, g1, be1, g5, be5):
    ei5 = edge_index_in.reshape(2, NW, NG, G, CHUNK)
    zeros = jnp.asarray(_ZEROS)

    sc_agg = _sc_aggregate_fn()
    p0, p1 = sc_agg(x_in, ei5, zeros)
    hid = _tc_call(p0, p1,
                   W1.T, b1[None, :], W2.T, b2[None, :],
                   g1[None, :], be1[None, :])

    q0, q1 = sc_agg(hid, ei5, zeros)
    W4p = jnp.pad(W4, ((0, D - W4.shape[0]), (0, 0)))
    b4p = jnp.pad(b4, (0, D - b4.shape[0]))
    g5p = jnp.pad(g5, (0, D - g5.shape[0]))
    be5p = jnp.pad(be5, (0, D - be5.shape[0]))
    h2 = _tc_call2(q0, q1,
                   W3.T, b3[None, :], W4p.T, b4p[None, :],
                   g5p[None, :], be5p[None, :])
    return (h2, hid)


# speedup vs baseline: 1.0446x; 1.0446x over previous
import functools

import numpy as np

import jax
import jax.numpy as jnp
from jax import lax
from jax.experimental import pallas as pl
from jax.experimental.pallas import tpu as pltpu
from jax.experimental.pallas import tpu_sc as plsc

N = 10000
E = 320000
D = 128
NC = 2
NS = 16
NW = NC * NS
CHUNK = 128
G = 8
NG = 10
CH = NG * G
EPT = CH * CHUNK
E_PAD = NW * EPT
AR = 10240
RPT = AR // NS
TAIL = N - (NS - 1) * RPT

@functools.cache
def _sc_aggregate_fn():
    mesh = plsc.VectorSubcoreMesh(core_axis_name="c", subcore_axis_name="s")

    @functools.partial(
        pl.kernel,
        mesh=mesh,
        out_type=(jax.ShapeDtypeStruct((AR, D), jnp.float32),
                  jax.ShapeDtypeStruct((AR, D), jnp.float32)),
        scratch_types=[
            pltpu.VMEM((2, G, CHUNK), jnp.int32),
            pltpu.VMEM((2, G, CHUNK), jnp.int32),
            pltpu.VMEM((CHUNK, D), jnp.float32),
            pltpu.VMEM((CHUNK, D), jnp.float32),
            pltpu.VMEM_SHARED((AR, D), jnp.float32),
            pltpu.SemaphoreType.DMA,
            pltpu.SemaphoreType.DMA,
            pltpu.SemaphoreType.DMA,
            pltpu.SemaphoreType.DMA,
        ],
    )
    def _sc_aggregate(table_hbm, ei_hbm, zeros_hbm, out0_hbm, out1_hbm,
                      src_idx, dst_idx, rows0, rows1, acc,
                      gsem0, gsem1, isem_s, isem_d):
        c = lax.axis_index("c")
        s = lax.axis_index("s")
        wid = c * NS + s
        base = s * RPT

        pltpu.sync_copy(ei_hbm.at[0, wid, 0], src_idx.at[0])
        pltpu.sync_copy(ei_hbm.at[1, wid, 0], dst_idx.at[0])
        pltpu.async_copy(table_hbm.at[src_idx.at[0, 0]], rows0, gsem0)
        pltpu.async_copy(table_hbm.at[src_idx.at[0, 1]], rows1, gsem1)

        @pl.when(jnp.logical_and(c == 0, s < NS - 1))
        def _():
            pltpu.sync_copy(table_hbm.at[pl.ds(base, RPT)],
                            acc.at[pl.ds(base, RPT)])

        @pl.when(jnp.logical_and(c == 0, s == NS - 1))
        def _():
            pltpu.sync_copy(table_hbm.at[pl.ds(base, TAIL)],
                            acc.at[pl.ds(base, TAIL)])
            pltpu.sync_copy(zeros_hbm.at[pl.ds(0, RPT - TAIL)],
                            acc.at[pl.ds(base + TAIL, RPT - TAIL)])

        @pl.when(c == 1)
        def _():
            pltpu.sync_copy(zeros_hbm, acc.at[pl.ds(base, RPT)])

        plsc.subcore_barrier()

        def outer(g, carry):
            pg = g % 2
            png = (g + 1) % 2

            @pl.when(g + 1 < NG)
            def _():
                pltpu.async_copy(ei_hbm.at[0, wid, g + 1], src_idx.at[png],
                                 isem_s)
                pltpu.async_copy(ei_hbm.at[1, wid, g + 1], dst_idx.at[png],
                                 isem_d)

            for k in range(G):
                rows, gsem = (rows0, gsem0) if k % 2 == 0 else (rows1, gsem1)
                pltpu.make_async_copy(
                    table_hbm.at[src_idx.at[pg, k]], rows, gsem).wait()
                pltpu.sync_copy(rows, acc.at[dst_idx.at[pg, k]], add=True)
                if k < G - 2:
                    pltpu.async_copy(
                        table_hbm.at[src_idx.at[pg, k + 2]], rows, gsem)
                else:
                    @pl.when(g + 1 < NG)
                    def _(k=k, rows=rows, gsem=gsem, png=png):
                        if k == G - 2:
                            pltpu.make_async_copy(
                                ei_hbm.at[0, wid, g + 1], src_idx.at[png],
                                isem_s).wait()
                            pltpu.make_async_copy(
                                ei_hbm.at[1, wid, g + 1], dst_idx.at[png],
                                isem_d).wait()
                        pltpu.async_copy(
                            table_hbm.at[src_idx.at[png, k - (G - 2)]],
                            rows, gsem)

            return carry

        lax.fori_loop(0, NG, outer, 0)
        plsc.subcore_barrier()

        @pl.when(c == 0)
        def _():
            pltpu.sync_copy(acc.at[pl.ds(base, RPT)],
                            out0_hbm.at[pl.ds(base, RPT)])

        @pl.when(c == 1)
        def _():
            pltpu.sync_copy(acc.at[pl.ds(base, RPT)],
                            out1_hbm.at[pl.ds(base, RPT)])

    return _sc_aggregate


def _tc_mlp_bn(p0_ref, p1_ref, wa_ref, ba_ref, wb_ref, bb_ref, g_ref, be_ref,
               out_ref):
    h = p0_ref[...][:N] + p1_ref[...][:N]
    h = jnp.maximum(jnp.dot(h, wa_ref[...],
                            preferred_element_type=jnp.float32) + ba_ref[...], 0.0)
    h = jnp.dot(h, wb_ref[...], preferred_element_type=jnp.float32) + bb_ref[...]
    h = jnp.maximum(h, 0.0)
    mean = jnp.mean(h, axis=0, keepdims=True)
    ctr = h - mean
    var = jnp.mean(ctr * ctr, axis=0, keepdims=True)
    out_ref[...] = ctr * lax.rsqrt(var + 1e-5) * g_ref[...] + be_ref[...]


_tc_call = pl.pallas_call(
    _tc_mlp_bn,
    out_shape=jax.ShapeDtypeStruct((N, D), jnp.float32),
)


def _tc_mlp_bn_narrow(p0_ref, p1_ref, wa_ref, ba_ref, wb_ref, bb_ref, g_ref,
                      be_ref, out_ref):
    h = p0_ref[...][:N] + p1_ref[...][:N]
    h = jnp.maximum(jnp.dot(h, wa_ref[...],
                            preferred_element_type=jnp.float32) + ba_ref[...], 0.0)
    h = jnp.dot(h, wb_ref[...], preferred_element_type=jnp.float32) + bb_ref[...]
    h = jnp.maximum(h, 0.0)
    mean = jnp.mean(h, axis=0, keepdims=True)
    ctr = h - mean
    var = jnp.mean(ctr * ctr, axis=0, keepdims=True)
    out = ctr * lax.rsqrt(var + 1e-5) * g_ref[...] + be_ref[...]
    out_ref[...] = out[:, : out_ref.shape[1]]


_tc_call2 = pl.pallas_call(
    _tc_mlp_bn_narrow,
    out_shape=jax.ShapeDtypeStruct((N, 2), jnp.float32),
)


_PAD_EI = np.stack([np.arange(E_PAD - E, dtype=np.int32) % N,
                    N + np.arange(E_PAD - E, dtype=np.int32) % (AR - N)])
_ZEROS = np.zeros((RPT, D), np.float32)


def kernel(x_in, edge_index_in, W1, b1, W2, b2, W3, b3, W4, b4, g1, be1, g5, be5):
    ei5 = jnp.concatenate([edge_index_in, jnp.asarray(_PAD_EI)],
                          axis=1).reshape(2, NW, NG, G, CHUNK)
    zeros = jnp.asarray(_ZEROS)

    sc_agg = _sc_aggregate_fn()
    p0, p1 = sc_agg(x_in, ei5, zeros)
    hid = _tc_call(p0, p1,
                   W1.T, b1[None, :], W2.T, b2[None, :],
                   g1[None, :], be1[None, :])

    q0, q1 = sc_agg(hid, ei5, zeros)
    W4p = jnp.pad(W4, ((0, D - W4.shape[0]), (0, 0)))
    b4p = jnp.pad(b4, (0, D - b4.shape[0]))
    g5p = jnp.pad(g5, (0, D - g5.shape[0]))
    be5p = jnp.pad(be5, (0, D - be5.shape[0]))
    h2 = _tc_call2(q0, q1,
                   W3.T, b3[None, :], W4p.T, b4p[None, :],
                   g5p[None, :], be5p[None, :])
    return (h2, hid)


# speedup vs baseline: 1.0764x; 1.0304x over previous
import functools

import numpy as np

import jax
import jax.numpy as jnp
from jax import lax
from jax.experimental import pallas as pl
from jax.experimental.pallas import tpu as pltpu
from jax.experimental.pallas import tpu_sc as plsc

N = 10000
E = 320000
D = 128
NC = 2
NS = 16
NW = NC * NS
CHUNK = 112
G = 6
NG = 15
CH = NG * G
EPT = CH * CHUNK
E_PAD = NW * EPT
AR = 10112
RPT = AR // NS
TAIL = N - (NS - 1) * RPT

@functools.cache
def _sc_aggregate_fn():
    mesh = plsc.VectorSubcoreMesh(core_axis_name="c", subcore_axis_name="s")

    @functools.partial(
        pl.kernel,
        mesh=mesh,
        out_type=(jax.ShapeDtypeStruct((AR, D), jnp.float32),
                  jax.ShapeDtypeStruct((AR, D), jnp.float32)),
        scratch_types=[
            pltpu.VMEM((2, G, CHUNK), jnp.int32),
            pltpu.VMEM((2, G, CHUNK), jnp.int32),
            pltpu.VMEM((CHUNK, D), jnp.float32),
            pltpu.VMEM((CHUNK, D), jnp.float32),
            pltpu.VMEM((CHUNK, D), jnp.float32),
            pltpu.VMEM_SHARED((AR, D), jnp.float32),
            pltpu.SemaphoreType.DMA,
            pltpu.SemaphoreType.DMA,
            pltpu.SemaphoreType.DMA,
            pltpu.SemaphoreType.DMA,
            pltpu.SemaphoreType.DMA,
            pltpu.SemaphoreType.DMA,
            pltpu.SemaphoreType.DMA,
            pltpu.SemaphoreType.DMA,
        ],
    )
    def _sc_aggregate(table_hbm, ei_hbm, zeros_hbm, out0_hbm, out1_hbm,
                      src_idx, dst_idx, rows0, rows1, rows2, acc,
                      gsem0, gsem1, gsem2, ssem0, ssem1, ssem2,
                      isem_s, isem_d):
        c = lax.axis_index("c")
        s = lax.axis_index("s")
        wid = c * NS + s
        base = s * RPT

        pltpu.sync_copy(ei_hbm.at[0, wid, 0], src_idx.at[0])
        pltpu.sync_copy(ei_hbm.at[1, wid, 0], dst_idx.at[0])
        pltpu.async_copy(table_hbm.at[src_idx.at[0, 0]], rows0, gsem0)
        pltpu.async_copy(table_hbm.at[src_idx.at[0, 1]], rows1, gsem1)

        @pl.when(jnp.logical_and(c == 0, s < NS - 1))
        def _():
            pltpu.sync_copy(table_hbm.at[pl.ds(base, RPT)],
                            acc.at[pl.ds(base, RPT)])

        @pl.when(jnp.logical_and(c == 0, s == NS - 1))
        def _():
            pltpu.sync_copy(table_hbm.at[pl.ds(base, TAIL)],
                            acc.at[pl.ds(base, TAIL)])
            pltpu.sync_copy(zeros_hbm.at[pl.ds(0, RPT - TAIL)],
                            acc.at[pl.ds(base + TAIL, RPT - TAIL)])

        @pl.when(c == 1)
        def _():
            pltpu.sync_copy(zeros_hbm, acc.at[pl.ds(base, RPT)])

        plsc.subcore_barrier()

        RB = (rows0, rows1, rows2)
        GS = (gsem0, gsem1, gsem2)
        SS = (ssem0, ssem1, ssem2)

        def outer(g, carry):
            pg = g % 2
            png = (g + 1) % 2

            @pl.when(g + 1 < NG)
            def _():
                pltpu.async_copy(ei_hbm.at[0, wid, g + 1], src_idx.at[png],
                                 isem_s)
                pltpu.async_copy(ei_hbm.at[1, wid, g + 1], dst_idx.at[png],
                                 isem_d)

            for k in range(G):
                b = k % 3
                bp = (k + 2) % 3
                pltpu.make_async_copy(
                    table_hbm.at[src_idx.at[pg, k]], RB[b], GS[b]).wait()
                pltpu.async_copy(RB[b], acc.at[dst_idx.at[pg, k]], SS[b],
                                 add=True)

                def _wait_prev(bp=bp):
                    pltpu.make_async_copy(
                        RB[bp], acc.at[dst_idx.at[0, 0]], SS[bp]).wait()

                if k == 0:
                    @pl.when(g == 0)
                    def _():
                        pltpu.async_copy(
                            table_hbm.at[src_idx.at[pg, 2]], RB[bp], GS[bp])

                    @pl.when(g > 0)
                    def _():
                        _wait_prev()
                        pltpu.async_copy(
                            table_hbm.at[src_idx.at[pg, 2]], RB[bp], GS[bp])
                elif k <= G - 3:
                    _wait_prev()
                    pltpu.async_copy(
                        table_hbm.at[src_idx.at[pg, k + 2]], RB[bp], GS[bp])
                else:
                    _wait_prev()

                    @pl.when(g + 1 < NG)
                    def _(k=k, bp=bp, png=png):
                        if k == G - 2:
                            pltpu.make_async_copy(
                                ei_hbm.at[0, wid, g + 1], src_idx.at[png],
                                isem_s).wait()
                            pltpu.make_async_copy(
                                ei_hbm.at[1, wid, g + 1], dst_idx.at[png],
                                isem_d).wait()
                        pltpu.async_copy(
                            table_hbm.at[src_idx.at[png, k - (G - 2)]],
                            RB[bp], GS[bp])

            return carry

        lax.fori_loop(0, NG, outer, 0)
        pltpu.make_async_copy(RB[(CH - 1) % 3], acc.at[dst_idx.at[0, 0]],
                              SS[(CH - 1) % 3]).wait()
        plsc.subcore_barrier()

        @pl.when(c == 0)
        def _():
            pltpu.sync_copy(acc.at[pl.ds(base, RPT)],
                            out0_hbm.at[pl.ds(base, RPT)])

        @pl.when(c == 1)
        def _():
            pltpu.sync_copy(acc.at[pl.ds(base, RPT)],
                            out1_hbm.at[pl.ds(base, RPT)])

    return _sc_aggregate


def _tc_mlp_bn(p0_ref, p1_ref, wa_ref, ba_ref, wb_ref, bb_ref, g_ref, be_ref,
               out_ref):
    h = p0_ref[...][:N] + p1_ref[...][:N]
    h = jnp.maximum(jnp.dot(h, wa_ref[...],
                            preferred_element_type=jnp.float32) + ba_ref[...], 0.0)
    h = jnp.dot(h, wb_ref[...], preferred_element_type=jnp.float32) + bb_ref[...]
    h = jnp.maximum(h, 0.0)
    mean = jnp.mean(h, axis=0, keepdims=True)
    ctr = h - mean
    var = jnp.mean(ctr * ctr, axis=0, keepdims=True)
    out_ref[...] = ctr * lax.rsqrt(var + 1e-5) * g_ref[...] + be_ref[...]


_tc_call = pl.pallas_call(
    _tc_mlp_bn,
    out_shape=jax.ShapeDtypeStruct((N, D), jnp.float32),
)


def _tc_mlp_bn_narrow(p0_ref, p1_ref, wa_ref, ba_ref, wb_ref, bb_ref, g_ref,
                      be_ref, out_ref):
    h = p0_ref[...][:N] + p1_ref[...][:N]
    h = jnp.maximum(jnp.dot(h, wa_ref[...],
                            preferred_element_type=jnp.float32) + ba_ref[...], 0.0)
    h = jnp.dot(h, wb_ref[...], preferred_element_type=jnp.float32) + bb_ref[...]
    h = jnp.maximum(h, 0.0)
    mean = jnp.mean(h, axis=0, keepdims=True)
    ctr = h - mean
    var = jnp.mean(ctr * ctr, axis=0, keepdims=True)
    out = ctr * lax.rsqrt(var + 1e-5) * g_ref[...] + be_ref[...]
    out_ref[...] = out[:, : out_ref.shape[1]]


_tc_call2 = pl.pallas_call(
    _tc_mlp_bn_narrow,
    out_shape=jax.ShapeDtypeStruct((N, 2), jnp.float32),
)


_PAD_EI = np.stack([np.arange(E_PAD - E, dtype=np.int32) % N,
                    N + np.arange(E_PAD - E, dtype=np.int32) % (AR - N)])
_ZEROS = np.zeros((RPT, D), np.float32)


def kernel(x_in, edge_index_in, W1, b1, W2, b2, W3, b3, W4, b4, g1, be1, g5, be5):
    ei5 = jnp.concatenate([edge_index_in, jnp.asarray(_PAD_EI)],
                          axis=1).reshape(2, NW, NG, G, CHUNK)
    zeros = jnp.asarray(_ZEROS)

    sc_agg = _sc_aggregate_fn()
    p0, p1 = sc_agg(x_in, ei5, zeros)
    hid = _tc_call(p0, p1,
                   W1.T, b1[None, :], W2.T, b2[None, :],
                   g1[None, :], be1[None, :])

    q0, q1 = sc_agg(hid, ei5, zeros)
    W4p = jnp.pad(W4, ((0, D - W4.shape[0]), (0, 0)))
    b4p = jnp.pad(b4, (0, D - b4.shape[0]))
    g5p = jnp.pad(g5, (0, D - g5.shape[0]))
    be5p = jnp.pad(be5, (0, D - be5.shape[0]))
    h2 = _tc_call2(q0, q1,
                   W3.T, b3[None, :], W4p.T, b4p[None, :],
                   g5p[None, :], be5p[None, :])
    return (h2, hid)
